# Initial kernel scaffold; baseline (speedup 1.0000x reference)
import jax
import jax.numpy as jnp
from jax.experimental import pallas as pl


def kernel(X, pref_a, pref_b):
    raise NotImplementedError("write your pallas kernel here")


# speedup vs baseline: 1.3122x; 1.3122x over previous
import jax
import jax.numpy as jnp
from jax import lax
from jax.experimental import pallas as pl
from jax.experimental.pallas import tpu as pltpu
from jax.experimental.pallas import tpu_sc as plsc

B = 16384
D = 128
L = 16

_info = plsc.get_sparse_core_info()
_NC = _info.num_cores
_NW = _NC * _info.num_subcores
ROWS_PER_W = B // _NW
CHUNK = 256
NCHUNK = ROWS_PER_W // CHUNK


def _sc_body(x_hbm, a_hbm, b_hbm, out_hbm,
             idx_a, idx_b, buf_a, buf_b, sem_a, sem_b):
    wid = lax.axis_index("s") * _NC + lax.axis_index("c")
    base = wid * ROWS_PER_W
    for c in range(NCHUNK):
        off = base + c * CHUNK
        pltpu.sync_copy(a_hbm.at[pl.ds(off, CHUNK)], idx_a)
        pltpu.sync_copy(b_hbm.at[pl.ds(off, CHUNK)], idx_b)
        cp_a = pltpu.async_copy(x_hbm.at[idx_a], buf_a, sem_a)
        cp_b = pltpu.async_copy(x_hbm.at[idx_b], buf_b, sem_b)
        cp_a.wait()
        cp_b.wait()

        def row(i, carry):
            for j in range(D // L):
                sl = pl.ds(j * L, L)
                buf_b[i, sl] = buf_b[i, sl] - buf_a[i, sl]
            return carry

        lax.fori_loop(0, CHUNK, row, 0)
        pltpu.sync_copy(buf_b, out_hbm.at[pl.ds(off, CHUNK)])


def kernel(X, pref_a, pref_b):
    mesh = plsc.VectorSubcoreMesh(core_axis_name="c", subcore_axis_name="s")
    k = pl.kernel(
        _sc_body,
        out_type=jax.ShapeDtypeStruct((B, D), jnp.float32),
        mesh=mesh,
        scratch_types=[
            pltpu.VMEM((CHUNK,), jnp.int32),
            pltpu.VMEM((CHUNK,), jnp.int32),
            pltpu.VMEM((CHUNK, D), jnp.float32),
            pltpu.VMEM((CHUNK, D), jnp.float32),
            pltpu.SemaphoreType.DMA,
            pltpu.SemaphoreType.DMA,
        ],
    )
    return k(X, pref_a.astype(jnp.int32), pref_b.astype(jnp.int32))


# speedup vs baseline: 1.4669x; 1.1179x over previous
import jax
import jax.numpy as jnp
from jax import lax
from jax.experimental import pallas as pl
from jax.experimental.pallas import tpu as pltpu
from jax.experimental.pallas import tpu_sc as plsc

B = 16384
D = 128
L = 16

_info = plsc.get_sparse_core_info()
_NC = _info.num_cores
_NW = _NC * _info.num_subcores
ROWS_PER_W = B // _NW
CHUNK = 128
NCH = ROWS_PER_W // CHUNK


def _sc_body(x_hbm, a_hbm, b_hbm, out_hbm,
             idx_a, idx_b, buf_a, buf_b, buf_o, sem_i, sem_g, sem_w):
    wid = lax.axis_index("s") * _NC + lax.axis_index("c")
    base = wid * ROWS_PER_W

    ci_a = pltpu.async_copy(a_hbm.at[pl.ds(base, ROWS_PER_W)], idx_a, sem_i)
    ci_b = pltpu.async_copy(b_hbm.at[pl.ds(base, ROWS_PER_W)], idx_b, sem_i)
    ci_a.wait()
    ci_b.wait()

    def start_gathers(c):
        s = c % 2
        isl = pl.ds(c * CHUNK, CHUNK)
        ha = pltpu.async_copy(x_hbm.at[idx_a.at[isl]], buf_a.at[s], sem_g.at[s])
        hb = pltpu.async_copy(x_hbm.at[idx_b.at[isl]], buf_b.at[s], sem_g.at[s])
        return ha, hb

    gh = [None, None]
    wh = [None, None]
    gh[0] = start_gathers(0)
    gh[1] = start_gathers(1)

    for c in range(NCH):
        s = c % 2
        gh[s][0].wait()
        gh[s][1].wait()
        if wh[s] is not None:
            wh[s].wait()

        def row(i, carry):
            for j in range(D // L):
                sl = pl.ds(j * L, L)
                buf_o[s, i, sl] = buf_b[s, i, sl] - buf_a[s, i, sl]
            return carry

        lax.fori_loop(0, CHUNK, row, 0)

        wh[s] = pltpu.async_copy(
            buf_o.at[s], out_hbm.at[pl.ds(base + c * CHUNK, CHUNK)], sem_w.at[s])
        if c + 2 < NCH:
            gh[s] = start_gathers(c + 2)

    for s in range(2):
        if wh[s] is not None:
            wh[s].wait()


def kernel(X, pref_a, pref_b):
    mesh = plsc.VectorSubcoreMesh(core_axis_name="c", subcore_axis_name="s")
    k = pl.kernel(
        _sc_body,
        out_type=jax.ShapeDtypeStruct((B, D), jnp.float32),
        mesh=mesh,
        scratch_types=[
            pltpu.VMEM((ROWS_PER_W,), jnp.int32),
            pltpu.VMEM((ROWS_PER_W,), jnp.int32),
            pltpu.VMEM((2, CHUNK, D), jnp.float32),
            pltpu.VMEM((2, CHUNK, D), jnp.float32),
            pltpu.VMEM((2, CHUNK, D), jnp.float32),
            pltpu.SemaphoreType.DMA,
            pltpu.SemaphoreType.DMA((2,)),
            pltpu.SemaphoreType.DMA((2,)),
        ],
    )
    return k(X, pref_a.astype(jnp.int32), pref_b.astype(jnp.int32))


# speedup vs baseline: 1.5414x; 1.0508x over previous
import jax
import jax.numpy as jnp
from jax import lax
from jax.experimental import pallas as pl
from jax.experimental.pallas import tpu as pltpu
from jax.experimental.pallas import tpu_sc as plsc

B = 16384
D = 128
L = 16

_info = plsc.get_sparse_core_info()
_NC = _info.num_cores
_NW = _NC * _info.num_subcores
ROWS_PER_W = B // _NW
CHUNK = 128
NCH = ROWS_PER_W // CHUNK
RPI = 2


def _sc_body(x_hbm, a_hbm, b_hbm, out_hbm, idx_a, idx_b, buf, sem_i, sem_g, sem_w):
    wid = lax.axis_index("s") * _NC + lax.axis_index("c")
    base = wid * ROWS_PER_W

    ci_a = pltpu.async_copy(a_hbm.at[pl.ds(base, ROWS_PER_W)], idx_a, sem_i)
    ci_b = pltpu.async_copy(b_hbm.at[pl.ds(base, ROWS_PER_W)], idx_b, sem_i)
    ci_a.wait()
    ci_b.wait()

    ga = [pltpu.async_copy(x_hbm.at[idx_a.at[pl.ds(c * CHUNK, CHUNK)]],
                           buf.at[c], sem_g.at[c])
          for c in range(NCH)]
    gb = [None] * NCH
    wb = [None] * NCH

    for c in range(NCH):
        ga[c].wait()

        def rows(t, carry):
            for r in range(RPI):
                for j in range(D // L):
                    i = t * RPI + r
                    sl = pl.ds(j * L, L)
                    buf[c, i, sl] = -buf[c, i, sl]
            return carry

        lax.fori_loop(0, CHUNK // RPI, rows, 0)

        gb[c] = pltpu.async_copy(x_hbm.at[idx_b.at[pl.ds(c * CHUNK, CHUNK)]],
                                 buf.at[c], sem_g.at[c], add=True)
        if c > 0:
            gb[c - 1].wait()
            wb[c - 1] = pltpu.async_copy(
                buf.at[c - 1], out_hbm.at[pl.ds(base + (c - 1) * CHUNK, CHUNK)],
                sem_w.at[c - 1])

    gb[NCH - 1].wait()
    wb[NCH - 1] = pltpu.async_copy(
        buf.at[NCH - 1], out_hbm.at[pl.ds(base + (NCH - 1) * CHUNK, CHUNK)],
        sem_w.at[NCH - 1])
    for c in range(NCH):
        wb[c].wait()


def kernel(X, pref_a, pref_b):
    mesh = plsc.VectorSubcoreMesh(core_axis_name="c", subcore_axis_name="s")
    k = pl.kernel(
        _sc_body,
        out_type=jax.ShapeDtypeStruct((B, D), jnp.float32),
        mesh=mesh,
        scratch_types=[
            pltpu.VMEM((ROWS_PER_W,), jnp.int32),
            pltpu.VMEM((ROWS_PER_W,), jnp.int32),
            pltpu.VMEM((NCH, CHUNK, D), jnp.float32),
            pltpu.SemaphoreType.DMA,
            pltpu.SemaphoreType.DMA((NCH,)),
            pltpu.SemaphoreType.DMA((NCH,)),
        ],
    )
    return k(X, pref_a.astype(jnp.int32), pref_b.astype(jnp.int32))


# speedup vs baseline: 1.5563x; 1.0097x over previous
import jax
import jax.numpy as jnp
from jax import lax
from jax.experimental import pallas as pl
from jax.experimental.pallas import tpu as pltpu
from jax.experimental.pallas import tpu_sc as plsc

B = 16384
D = 128
L = 16

_info = plsc.get_sparse_core_info()
_NC = _info.num_cores
_NW = _NC * _info.num_subcores
ROWS_PER_W = B // _NW
CHUNK = 256
NCH = ROWS_PER_W // CHUNK
RPI = 2


def _sc_body(x_hbm, a_hbm, b_hbm, out_hbm, idx_a, idx_b, buf, sem_i, sem_g, sem_w):
    wid = lax.axis_index("s") * _NC + lax.axis_index("c")
    base = wid * ROWS_PER_W

    ci_a = pltpu.async_copy(a_hbm.at[pl.ds(base, ROWS_PER_W)], idx_a, sem_i)
    ci_b = pltpu.async_copy(b_hbm.at[pl.ds(base, ROWS_PER_W)], idx_b, sem_i)
    ci_a.wait()
    ci_b.wait()

    ga = [pltpu.async_copy(x_hbm.at[idx_a.at[pl.ds(c * CHUNK, CHUNK)]],
                           buf.at[c], sem_g.at[c])
          for c in range(NCH)]
    gb = [None] * NCH
    wb = [None] * NCH

    for c in range(NCH):
        ga[c].wait()

        def rows(t, carry):
            for r in range(RPI):
                for j in range(D // L):
                    i = t * RPI + r
                    sl = pl.ds(j * L, L)
                    buf[c, i, sl] = -buf[c, i, sl]
            return carry

        lax.fori_loop(0, CHUNK // RPI, rows, 0)

        gb[c] = pltpu.async_copy(x_hbm.at[idx_b.at[pl.ds(c * CHUNK, CHUNK)]],
                                 buf.at[c], sem_g.at[c], add=True)
        if c > 0:
            gb[c - 1].wait()
            wb[c - 1] = pltpu.async_copy(
                buf.at[c - 1], out_hbm.at[pl.ds(base + (c - 1) * CHUNK, CHUNK)],
                sem_w.at[c - 1])

    gb[NCH - 1].wait()
    wb[NCH - 1] = pltpu.async_copy(
        buf.at[NCH - 1], out_hbm.at[pl.ds(base + (NCH - 1) * CHUNK, CHUNK)],
        sem_w.at[NCH - 1])
    for c in range(NCH):
        wb[c].wait()


def kernel(X, pref_a, pref_b):
    mesh = plsc.VectorSubcoreMesh(core_axis_name="c", subcore_axis_name="s")
    k = pl.kernel(
        _sc_body,
        out_type=jax.ShapeDtypeStruct((B, D), jnp.float32),
        mesh=mesh,
        scratch_types=[
            pltpu.VMEM((ROWS_PER_W,), jnp.int32),
            pltpu.VMEM((ROWS_PER_W,), jnp.int32),
            pltpu.VMEM((NCH, CHUNK, D), jnp.float32),
            pltpu.SemaphoreType.DMA,
            pltpu.SemaphoreType.DMA((NCH,)),
            pltpu.SemaphoreType.DMA((NCH,)),
        ],
    )
    return k(X, pref_a.astype(jnp.int32), pref_b.astype(jnp.int32))
